# baseline (speedup 1.0000x reference)
import functools

import jax
import jax.numpy as jnp
from jax import lax
from jax.experimental import pallas as pl
from jax.experimental.pallas import tpu as pltpu
from jax.experimental.pallas import tpu_sc as plsc

B = 16384
D = 64
N_COND = 8
NC = 2
NS = 16
NW = NC * NS
BPW = B // NW
CHUNK = 128
NCHUNK = BPW // CHUNK

_sc_mesh = plsc.VectorSubcoreMesh(core_axis_name="c", subcore_axis_name="s")


@functools.partial(
    pl.kernel,
    out_type=jax.ShapeDtypeStruct((B, D), jnp.float32),
    mesh=_sc_mesh,
    scratch_types=[
        pltpu.VMEM((NCHUNK, CHUNK), jnp.int32),
        pltpu.VMEM((BPW, D), jnp.float32),
        pltpu.SemaphoreType.DMA,
    ],
    compiler_params=pltpu.CompilerParams(use_tc_tiling_on_sc=False),
)
def _sc_gather(table_hbm, xr_hbm, out_hbm, idx_v, rows_v, sem):
    wid = lax.axis_index("s") * NC + lax.axis_index("c")
    pltpu.sync_copy(xr_hbm.at[wid], idx_v)
    copies = [
        pltpu.async_copy(
            table_hbm.at[idx_v.at[j]],
            rows_v.at[pl.ds(j * CHUNK, CHUNK)],
            sem,
        )
        for j in range(NCHUNK)
    ]
    for cp in copies:
        cp.wait()
    pltpu.sync_copy(rows_v, out_hbm.at[pl.ds(wid * BPW, BPW)])


def _tc_body(emb_ref, c_ref, masks_ref, out_ref, mn_ref, en_ref):
    emb = emb_ref[...]
    cc = c_ref[...]
    m = jnp.maximum(masks_ref[...], 0.0)
    iota = lax.broadcasted_iota(jnp.int32, (1, N_COND), 1)
    onehot = (cc == iota).astype(jnp.float32)
    mask = lax.dot_general(
        onehot, m, (((1,), (0,)), ((), ())),
        preferred_element_type=jnp.float32,
    )
    masked = emb * mask
    rn = jnp.sqrt(jnp.sum(masked * masked, axis=1, keepdims=True))
    out_ref[...] = masked / jnp.maximum(rn, 1e-10)
    mn_ref[...] = jnp.reshape(jnp.sum(jnp.abs(mask)), (1, 1))
    en_ref[...] = jnp.reshape(jnp.sqrt(jnp.sum(emb * emb)), (1, 1))


_tc_call = pl.pallas_call(
    _tc_body,
    out_shape=[
        jax.ShapeDtypeStruct((B, D), jnp.float32),
        jax.ShapeDtypeStruct((1, 1), jnp.float32),
        jax.ShapeDtypeStruct((1, 1), jnp.float32),
    ],
)


def kernel(x, c, table, masks):
    xr = x.astype(jnp.int32).reshape(NW, NCHUNK, CHUNK)
    embedded = _sc_gather(table, xr)
    out, mn, en = _tc_call(embedded, c.astype(jnp.int32).reshape(B, 1), masks)
    return out, mn.reshape(()), en.reshape(()), embedded


# speedup vs baseline: 1.6749x; 1.6749x over previous
import functools

import jax
import jax.numpy as jnp
from jax import lax
from jax.experimental import pallas as pl
from jax.experimental.pallas import tpu as pltpu
from jax.experimental.pallas import tpu_sc as plsc

B = 16384
D = 64
N_COND = 8
NC = 2
NS = 16
NW = NC * NS
BPW = B // NW

_sc_mesh = plsc.VectorSubcoreMesh(core_axis_name="c", subcore_axis_name="s")


@functools.partial(
    pl.kernel,
    out_type=jax.ShapeDtypeStruct((B, D), jnp.float32),
    mesh=_sc_mesh,
    scratch_types=[
        pltpu.VMEM((BPW,), jnp.int32),
        pltpu.VMEM((BPW, D), jnp.float32),
        pltpu.SemaphoreType.DMA,
    ],
)
def _sc_gather(table_hbm, xw_hbm, out_hbm, idx_v, rows_v, sem):
    wid = lax.axis_index("s") * NC + lax.axis_index("c")
    pltpu.sync_copy(xw_hbm.at[wid], idx_v)

    def _grp(g, _):
        base = g * 16
        vec = idx_v[pl.ds(base, 16)]
        for j in range(16):
            pltpu.async_copy(table_hbm.at[pl.ds(vec[j], 1)],
                             rows_v.at[pl.ds(base + j, 1)], sem)
        return 0

    lax.fori_loop(0, BPW // 16, _grp, 0)
    pltpu.make_async_copy(table_hbm.at[pl.ds(0, BPW)], rows_v, sem).wait()
    pltpu.sync_copy(rows_v, out_hbm.at[pl.ds(wid * BPW, BPW)])


def _tc_body(emb_ref, c_ref, masks_ref, out_ref, mn_ref, en_ref):
    emb = emb_ref[...]
    cc = c_ref[...]
    m = jnp.maximum(masks_ref[...], 0.0)
    iota = lax.broadcasted_iota(jnp.int32, (1, N_COND), 1)
    onehot = (cc == iota).astype(jnp.float32)
    mask = lax.dot_general(
        onehot, m, (((1,), (0,)), ((), ())),
        preferred_element_type=jnp.float32,
    )
    masked = emb * mask
    rn = jnp.sqrt(jnp.sum(masked * masked, axis=1, keepdims=True))
    out_ref[...] = masked / jnp.maximum(rn, 1e-10)
    mn_ref[...] = jnp.reshape(jnp.sum(jnp.abs(mask)), (1, 1))
    en_ref[...] = jnp.reshape(jnp.sqrt(jnp.sum(emb * emb)), (1, 1))


_tc_call = pl.pallas_call(
    _tc_body,
    out_shape=[
        jax.ShapeDtypeStruct((B, D), jnp.float32),
        jax.ShapeDtypeStruct((1, 1), jnp.float32),
        jax.ShapeDtypeStruct((1, 1), jnp.float32),
    ],
)


def kernel(x, c, table, masks):
    xw = x.astype(jnp.int32).reshape(NW, BPW)
    embedded = _sc_gather(table, xw)
    out, mn, en = _tc_call(embedded, c.astype(jnp.int32).reshape(B, 1), masks)
    return out, mn.reshape(()), en.reshape(()), embedded
